# Initial kernel scaffold; baseline (speedup 1.0000x reference)
import jax
import jax.numpy as jnp
from jax.experimental import pallas as pl


def kernel(boxes, scores):
    raise NotImplementedError("write your pallas kernel here")


# speedup vs baseline: 20.0804x; 20.0804x over previous
import functools

import jax
import jax.numpy as jnp
from jax import lax
from jax.experimental import pallas as pl
from jax.experimental.pallas import tpu as pltpu

N = 20000
NP = 20480
ROWS = NP // 128
MAX_DET = 100
CONF_THRES = 0.2
IOU_THRES = 0.4
NEG_INF = float("-inf")


def _nms_body(x1_ref, y1_ref, x2_ref, y2_ref, s_ref, out_ref, ms_ref):
    x1 = x1_ref[...]
    y1 = y1_ref[...]
    x2 = x2_ref[...]
    y2 = y2_ref[...]
    s = s_ref[...]

    dw = x2 - x1
    dh = y2 - y1
    above = s > CONF_THRES
    any_above = jnp.any(above)
    act = (above | jnp.logical_not(any_above)) & (dw >= 1.0) & (dh >= 1.0)
    ms_ref[...] = jnp.where(act, s, NEG_INF)
    area = dw * dh

    ridx = lax.broadcasted_iota(jnp.int32, (ROWS, 128), 0)
    cidx = lax.broadcasted_iota(jnp.int32, (ROWS, 128), 1)
    idx2d = ridx * 128 + cidx
    lane = lax.broadcasted_iota(jnp.int32, (1, 128), 1)

    def step(i, carry):
        kx1, ky1, kx2, ky2, ks, kv = carry
        ms = ms_ref[...]
        m = jnp.max(ms)
        valid = m > NEG_INF
        eq = ms == m
        j = jnp.min(jnp.where(eq, idx2d, jnp.int32(2**30)))
        onehot = idx2d == j
        zero = jnp.float32(0.0)
        bx1 = jnp.sum(jnp.where(onehot, x1, zero))
        by1 = jnp.sum(jnp.where(onehot, y1, zero))
        bx2 = jnp.sum(jnp.where(onehot, x2, zero))
        by2 = jnp.sum(jnp.where(onehot, y2, zero))
        ix1 = jnp.maximum(bx1, x1)
        iy1 = jnp.maximum(by1, y1)
        ix2 = jnp.minimum(bx2, x2)
        iy2 = jnp.minimum(by2, y2)
        inter = jnp.clip(ix2 - ix1, 0.0) * jnp.clip(iy2 - iy1, 0.0)
        barea = (bx2 - bx1) * (by2 - by1)
        iou = inter / (barea + area - inter + 1e-9)
        ms_ref[...] = jnp.where(iou > IOU_THRES, NEG_INF, ms)

        sel = (lane == i) & valid
        kx1 = kx1 + jnp.where(sel, bx1, zero)
        ky1 = ky1 + jnp.where(sel, by1, zero)
        kx2 = kx2 + jnp.where(sel, bx2, zero)
        ky2 = ky2 + jnp.where(sel, by2, zero)
        ks = ks + jnp.where(sel, m, zero)
        kv = kv + jnp.where(sel, jnp.float32(1.0), zero)
        return (kx1, ky1, kx2, ky2, ks, kv)

    z = jnp.zeros((1, 128), jnp.float32)
    kx1, ky1, kx2, ky2, ks, kv = lax.fori_loop(
        0, MAX_DET, step, (z, z, z, z, z, z))
    out_ref[...] = jnp.concatenate([kx1, ky1, kx2, ky2, ks, kv, z, z], axis=0)


@functools.partial(jax.jit, static_argnames=("interpret",))
def kernel(boxes, scores, interpret=False):
    boxes_p = jnp.pad(boxes, ((0, NP - N), (0, 0)))
    scores_p = jnp.pad(scores, (0, NP - N))
    x1 = boxes_p[:, 0].reshape(ROWS, 128)
    y1 = boxes_p[:, 1].reshape(ROWS, 128)
    x2 = boxes_p[:, 2].reshape(ROWS, 128)
    y2 = boxes_p[:, 3].reshape(ROWS, 128)
    sp = scores_p.reshape(ROWS, 128)
    out = pl.pallas_call(
        _nms_body,
        out_shape=jax.ShapeDtypeStruct((8, 128), jnp.float32),
        scratch_shapes=[pltpu.VMEM((ROWS, 128), jnp.float32)],
        interpret=interpret,
    )(x1, y1, x2, y2, sp)
    kept_boxes = jnp.stack(
        [out[0, :MAX_DET], out[1, :MAX_DET], out[2, :MAX_DET], out[3, :MAX_DET]],
        axis=1)
    kept_scores = out[4, :MAX_DET]
    selmask = out[5, :MAX_DET] > 0.5
    return kept_boxes, kept_scores, selmask
